# baseline (speedup 1.0000x reference)
import functools

import jax
import jax.numpy as jnp
from jax.experimental import pallas as pl
from jax.experimental.pallas import tpu as pltpu


def _gcn_body(nb, bm,
              x_ref, w1_ref, b1_ref, w2_ref, adj_ref,
              lsm_ref, out_ref,
              adj_scr, xw1_scr, hw2_scr):
    i = pl.program_id(0)

    @pl.when(i == 0)
    def _compute_xw1():
        xw1_scr[...] = jnp.dot(
            x_ref[...], w1_ref[...],
            preferred_element_type=jnp.float32).astype(jnp.bfloat16)

    ab = adj_ref[...].astype(jnp.bfloat16)
    adj_scr[pl.ds(i * bm, bm), :] = ab
    h = jnp.dot(ab, xw1_scr[...], preferred_element_type=jnp.float32)
    h = jnp.maximum(h + b1_ref[...], 0.0)
    hw2 = jnp.dot(h, w2_ref[...], preferred_element_type=jnp.float32)
    hw2_scr[pl.ds(i * bm, bm), :] = hw2.astype(jnp.bfloat16)
    lsm_ref[...] = hw2
    out_ref[...] = jnp.concatenate([hw2, hw2], axis=1)


def kernel(x, adj, W1, b1, W2, b2, W3, b3, encoder_type):
    n, nfeat = x.shape
    nhid = W1.shape[1]
    nclass = W2.shape[1]
    proj = W3.shape[1]

    bm = 512
    nb = n // bm
    b1r = b1.reshape(1, nhid)

    body = functools.partial(_gcn_body, nb, bm)

    lsm, out = pl.pallas_call(
        body,
        grid=(nb,),
        in_specs=[
            pl.BlockSpec((n, nfeat), lambda i: (0, 0)),
            pl.BlockSpec((nfeat, nhid), lambda i: (0, 0)),
            pl.BlockSpec((1, nhid), lambda i: (0, 0)),
            pl.BlockSpec((nhid, nclass), lambda i: (0, 0)),
            pl.BlockSpec((bm, n), lambda i: (i, 0)),
        ],
        out_specs=[
            pl.BlockSpec((bm, nclass), lambda i: (i, 0)),
            pl.BlockSpec((bm, proj), lambda i: (i, 0)),
        ],
        out_shape=[
            jax.ShapeDtypeStruct((n, nclass), jnp.float32),
            jax.ShapeDtypeStruct((n, proj), jnp.float32),
        ],
        scratch_shapes=[
            pltpu.VMEM((n, n), jnp.bfloat16),
            pltpu.VMEM((n, nhid), jnp.bfloat16),
            pltpu.VMEM((n, nclass), jnp.bfloat16),
        ],
        compiler_params=pltpu.CompilerParams(
            dimension_semantics=("arbitrary",),
            vmem_limit_bytes=100 * 1024 * 1024,
        ),
    )(x, W1, b1r, W2, adj)

    return (lsm, out)
